# Initial kernel scaffold; baseline (speedup 1.0000x reference)
import jax
import jax.numpy as jnp
from jax.experimental import pallas as pl


def kernel(x, adj, im_W0, im_b0, im_g0, im_bt0, im_W1, im_b1, Wg, bg, gng, gnb, Wl1, bl1, Wl2, bl2, Wo0, bo0, Wo1, bo1):
    raise NotImplementedError("write your pallas kernel here")


# speedup vs baseline: 5.6896x; 5.6896x over previous
import functools

import jax
import jax.numpy as jnp
from jax import lax
from jax.experimental import pallas as pl
from jax.experimental.pallas import tpu as pltpu
from jax.experimental.pallas import tpu_sc as plsc

N = 10000
E = 320000
D = 128
HID = 256
LH = 4
L = 4
EPS = 1e-5
SLOPE = 0.2

NC = 2
NS = 16
NW = NC * NS
CHUNK = 128
CHUNKS = 80
EP = NW * CHUNKS * CHUNK
NPAD = 10112
RPT = NPAD // NS

_mesh = plsc.VectorSubcoreMesh(
    core_axis_name="c", subcore_axis_name="s", num_cores=NC, num_subcores=NS
)


def _worker(c, s):
    return c * NS + s


def _sc_degree_body(dst_hbm, ones_hbm, zeros_hbm, out_hbm, deg_sh, dst_v, ones_v, sem):
    c = lax.axis_index("c")
    s = lax.axis_index("s")
    w = _worker(c, s)
    pltpu.async_copy(dst_hbm.at[pl.ds(w * CHUNKS, CHUNKS)], dst_v, sem).wait()
    pltpu.async_copy(ones_hbm, ones_v, sem).wait()
    pltpu.sync_copy(zeros_hbm.at[pl.ds(s * RPT, RPT)], deg_sh.at[pl.ds(s * RPT, RPT)])
    plsc.subcore_barrier()

    def body(j, carry):
        pltpu.sync_copy(ones_v, deg_sh.at[dst_v.at[j]], add=True)
        return carry

    lax.fori_loop(0, CHUNKS, body, 0)
    plsc.subcore_barrier()
    pltpu.sync_copy(deg_sh.at[pl.ds(s * RPT, RPT)], out_hbm.at[c, pl.ds(s * RPT, RPT)])


_sc_degree = pl.kernel(
    _sc_degree_body,
    out_type=jax.ShapeDtypeStruct((NC, NPAD, 16), jnp.float32),
    mesh=_mesh,
    scratch_types=[
        pltpu.VMEM_SHARED((NPAD, 16), jnp.float32),
        pltpu.VMEM((CHUNKS, CHUNK), jnp.int32),
        pltpu.VMEM((CHUNK, 16), jnp.float32),
        pltpu.SemaphoreType.DMA,
    ],
)


PH = CHUNKS // 2


def _sc_spmm_body(b_hbm, src_hbm, dst_hbm, zeros_hbm, out_hbm,
                  agg_sh, src_v, dst_v, rows_v, sem0, sem1):
    c = lax.axis_index("c")
    s = lax.axis_index("s")
    w = _worker(c, s)
    pltpu.sync_copy(zeros_hbm.at[pl.ds(s * RPT, RPT)], agg_sh.at[pl.ds(s * RPT, RPT)])
    plsc.subcore_barrier()

    for ph in range(CHUNKS // PH):
        base = w * CHUNKS + ph * PH
        pltpu.async_copy(src_hbm.at[pl.ds(base, PH)], src_v, sem0).wait()
        pltpu.async_copy(dst_hbm.at[pl.ds(base, PH)], dst_v, sem1).wait()

        pltpu.async_copy(b_hbm.at[src_v.at[0]], rows_v.at[0], sem0)
        pltpu.async_copy(b_hbm.at[src_v.at[1]], rows_v.at[1], sem1)

        def body(it, carry):
            j0 = it * 2
            j1 = j0 + 1
            pltpu.make_async_copy(b_hbm.at[src_v.at[j0]], rows_v.at[0], sem0).wait()
            pltpu.sync_copy(rows_v.at[0], agg_sh.at[dst_v.at[j0]], add=True)
            pltpu.async_copy(b_hbm.at[src_v.at[j0 + 2]], rows_v.at[0], sem0)
            pltpu.make_async_copy(b_hbm.at[src_v.at[j1]], rows_v.at[1], sem1).wait()
            pltpu.sync_copy(rows_v.at[1], agg_sh.at[dst_v.at[j1]], add=True)
            pltpu.async_copy(b_hbm.at[src_v.at[j1 + 2]], rows_v.at[1], sem1)
            return carry

        lax.fori_loop(0, PH // 2 - 1, body, 0)
        jl = PH - 2
        pltpu.make_async_copy(b_hbm.at[src_v.at[jl]], rows_v.at[0], sem0).wait()
        pltpu.sync_copy(rows_v.at[0], agg_sh.at[dst_v.at[jl]], add=True)
        pltpu.make_async_copy(b_hbm.at[src_v.at[jl + 1]], rows_v.at[1], sem1).wait()
        pltpu.sync_copy(rows_v.at[1], agg_sh.at[dst_v.at[jl + 1]], add=True)

    plsc.subcore_barrier()
    pltpu.sync_copy(agg_sh.at[pl.ds(s * RPT, RPT)], out_hbm.at[c, pl.ds(s * RPT, RPT)])


_sc_spmm = pl.kernel(
    _sc_spmm_body,
    out_type=jax.ShapeDtypeStruct((NC, NPAD, D), jnp.float32),
    mesh=_mesh,
    scratch_types=[
        pltpu.VMEM_SHARED((NPAD, D), jnp.float32),
        pltpu.VMEM((PH, CHUNK), jnp.int32),
        pltpu.VMEM((PH, CHUNK), jnp.int32),
        pltpu.VMEM((2, CHUNK, D), jnp.float32),
        pltpu.SemaphoreType.DMA,
        pltpu.SemaphoreType.DMA,
    ],
)


def _bn(t, g, b):
    mu = jnp.mean(t, axis=0, keepdims=True)
    var = jnp.mean((t - mu) * (t - mu), axis=0, keepdims=True)
    return (t - mu) * lax.rsqrt(var + EPS) * g + b


def _learner(inp, wl1, bl1, wl2, bl2):
    z = jnp.dot(inp, wl1, preferred_element_type=jnp.float32) + bl1
    z = jnp.where(z >= 0.0, z, SLOPE * z)
    u = jnp.dot(z, wl2, preferred_element_type=jnp.float32) + bl2
    return 1.0 / (1.0 + jnp.exp(-u))


def _tc_start_body(x, w0, b0, g0, bt0, w1, b1, wl1, bl1, wl2, bl2, wg0, degp,
                   h_out, dinv_out, invd_out, b_out, hself_out):
    t = jnp.dot(x[...], w0[...], preferred_element_type=jnp.float32) + b0[...]
    t = jnp.maximum(_bn(t, g0[...], bt0[...]), 0.0)
    h = jnp.dot(t, w1[...], preferred_element_type=jnp.float32) + b1[...]
    nw = _learner(h, wl1[...], bl1[...], wl2[...], bl2[...])
    h = h * nw
    h_out[...] = h
    deg = degp[0, :N, 0:1] + degp[1, :N, 0:1] + 1.0
    dinv = lax.rsqrt(deg)
    invd = 1.0 / deg
    dinv_out[...] = dinv
    invd_out[...] = invd
    hw = jnp.dot(h, wg0[...], preferred_element_type=jnp.float32)
    b_out[...] = hw * dinv
    hself_out[...] = hw * invd


_TC_PARAMS = pltpu.CompilerParams(vmem_limit_bytes=100 * 1024 * 1024)

_tc_start = pl.pallas_call(
    _tc_start_body,
    compiler_params=_TC_PARAMS,
    out_shape=[
        jax.ShapeDtypeStruct((N, D), jnp.float32),
        jax.ShapeDtypeStruct((N, 1), jnp.float32),
        jax.ShapeDtypeStruct((N, 1), jnp.float32),
        jax.ShapeDtypeStruct((N, D), jnp.float32),
        jax.ShapeDtypeStruct((N, D), jnp.float32),
    ],
)


def _tc_mid_body(aggp, hself, dinv, invd, fused, bg, g, bt, wl1, bl1, wl2, bl2,
                 wg_next, fused_out, b_out, hself_out):
    dv = dinv[...]
    agg = (aggp[0, :N, :] + aggp[1, :N, :]) * dv + hself[...] + bg[...]
    h = jnp.maximum(_bn(agg, g[...], bt[...]), 0.0)
    f = fused[...]
    nw = _learner(h + f, wl1[...], bl1[...], wl2[...], bl2[...])
    h = h * nw
    f = f + h
    fused_out[...] = f
    hw = jnp.dot(h, wg_next[...], preferred_element_type=jnp.float32)
    b_out[...] = hw * dv
    hself_out[...] = hw * invd[...]


_tc_mid = pl.pallas_call(
    _tc_mid_body,
    compiler_params=_TC_PARAMS,
    out_shape=[
        jax.ShapeDtypeStruct((N, D), jnp.float32),
        jax.ShapeDtypeStruct((N, D), jnp.float32),
        jax.ShapeDtypeStruct((N, D), jnp.float32),
    ],
)


def _tc_final_body(aggp, hself, dinv, fused, bg, g, bt, wl1, bl1, wl2, bl2,
                   wo0, bo0, wo1, bo1, o_out):
    agg = (aggp[0, :N, :] + aggp[1, :N, :]) * dinv[...] + hself[...] + bg[...]
    h = jnp.maximum(_bn(agg, g[...], bt[...]), 0.0)
    f = fused[...]
    nw = _learner(h + f, wl1[...], bl1[...], wl2[...], bl2[...])
    f = f + h * nw
    t = jnp.dot(f, wo0[...], preferred_element_type=jnp.float32) + bo0[...]
    t = jnp.maximum(t, 0.0)
    o_out[...] = jnp.dot(t, wo1[...], preferred_element_type=jnp.float32) + bo1[...]


_tc_final = pl.pallas_call(
    _tc_final_body,
    compiler_params=_TC_PARAMS,
    out_shape=jax.ShapeDtypeStruct((N, D), jnp.float32),
)


def kernel(x, adj, im_W0, im_b0, im_g0, im_bt0, im_W1, im_b1, Wg, bg, gng, gnb,
           Wl1, bl1, Wl2, bl2, Wo0, bo0, Wo1, bo1):
    src = adj[0]
    dst = adj[1]
    pad = EP - E
    srcp = jnp.concatenate([src, jnp.zeros((pad,), jnp.int32)]).reshape(NW * CHUNKS, CHUNK)
    dstp = jnp.concatenate(
        [dst, N + (jnp.arange(pad, dtype=jnp.int32) % 16)]
    ).reshape(NW * CHUNKS, CHUNK)

    ones16 = jnp.ones((CHUNK, 16), jnp.float32)
    zeros_deg = jnp.zeros((NPAD, 16), jnp.float32)
    zeros_feat = jnp.zeros((NPAD, D), jnp.float32)

    degp = _sc_degree(dstp, ones16, zeros_deg)

    r2 = lambda v: v.reshape(1, -1)
    h, dinv, invd, b, hself = _tc_start(
        x, im_W0, r2(im_b0), r2(im_g0), r2(im_bt0), im_W1, r2(im_b1),
        Wl1[0], r2(bl1[0]), Wl2[0], r2(bl2[0]), Wg[0], degp)

    fused = h
    for i in range(L):
        aggp = _sc_spmm(b, srcp, dstp, zeros_feat)
        if i < L - 1:
            fused, b, hself = _tc_mid(
                aggp, hself, dinv, invd, fused, r2(bg[i]), r2(gng[i]), r2(gnb[i]),
                Wl1[i + 1], r2(bl1[i + 1]), Wl2[i + 1], r2(bl2[i + 1]), Wg[i + 1])
        else:
            o = _tc_final(
                aggp, hself, dinv, fused, r2(bg[i]), r2(gng[i]), r2(gnb[i]),
                Wl1[i + 1], r2(bl1[i + 1]), Wl2[i + 1], r2(bl2[i + 1]),
                Wo0, r2(bo0), Wo1, r2(bo1))
    return o
